# baseline (speedup 1.0000x reference)
import functools

import jax
import jax.numpy as jnp
from jax import lax
from jax.experimental import pallas as pl
from jax.experimental.pallas import tpu as pltpu
from jax.experimental.pallas import tpu_sc as plsc

V = 100000
E = 16
B = 1024

_NC = 2
_NS = 16
_NW = _NC * _NS
_B_PER_W = B // _NW

TILE_V = 512
GRID_V = (V + TILE_V - 1) // TILE_V


def _sc_gather(idx, table):
    mesh = plsc.VectorSubcoreMesh(core_axis_name="c", subcore_axis_name="s")

    @functools.partial(
        pl.kernel,
        mesh=mesh,
        out_type=jax.ShapeDtypeStruct((B, E), jnp.float32),
        scratch_types=[
            pltpu.VMEM((_B_PER_W,), jnp.int32),
            pltpu.VMEM((_B_PER_W, E), jnp.float32),
            pltpu.SemaphoreType.DMA,
        ],
        compiler_params=pltpu.CompilerParams(use_tc_tiling_on_sc=False),
    )
    def gather_kernel(idx_hbm, table_hbm, out_hbm, idx_v, rows_v, sem):
        wid = lax.axis_index("s") * _NC + lax.axis_index("c")
        base = wid * _B_PER_W
        pltpu.sync_copy(idx_hbm.at[pl.ds(base, _B_PER_W)], idx_v)
        pltpu.async_copy(table_hbm.at[idx_v], rows_v, sem).wait()
        pltpu.sync_copy(rows_v, out_hbm.at[pl.ds(base, _B_PER_W)])

    return gather_kernel(idx, table)


def _proj_kernel(emb_ref, w_ref, b_ref, out_ref):
    acc = lax.dot_general(
        emb_ref[...],
        w_ref[...],
        dimension_numbers=(((1,), (1,)), ((), ())),
        preferred_element_type=jnp.float32,
    )
    out_ref[:, 0, :] = acc + b_ref[...]


def _tc_project(emb, w, b2):
    return pl.pallas_call(
        _proj_kernel,
        grid=(GRID_V,),
        in_specs=[
            pl.BlockSpec((B, E), lambda j: (0, 0)),
            pl.BlockSpec((TILE_V, E), lambda j: (j, 0)),
            pl.BlockSpec((1, TILE_V), lambda j: (0, j)),
        ],
        out_specs=pl.BlockSpec((B, 1, TILE_V), lambda j: (0, 0, j)),
        out_shape=jax.ShapeDtypeStruct((B, 1, V), jnp.float32),
    )(emb, w, b2)


def kernel(target, emb_table, W0, b0):
    idx = target.astype(jnp.int32)
    emb = _sc_gather(idx, emb_table)
    return _tc_project(emb, W0, b0.reshape(1, V))


# speedup vs baseline: 2.1227x; 2.1227x over previous
import functools

import jax
import jax.numpy as jnp
from jax import lax
from jax.experimental import pallas as pl
from jax.experimental.pallas import tpu as pltpu
from jax.experimental.pallas import tpu_sc as plsc

V = 100000
E = 16
B = 1024

_NC = 2
_NS = 16
_NW = _NC * _NS
_B_PER_W = B // _NW

TILE_V = 512
GRID_V = (V + TILE_V - 1) // TILE_V


def _sc_gather(idx, table):
    mesh = plsc.VectorSubcoreMesh(core_axis_name="c", subcore_axis_name="s")

    @functools.partial(
        pl.kernel,
        mesh=mesh,
        out_type=jax.ShapeDtypeStruct((B, E), jnp.float32),
        scratch_types=[
            pltpu.VMEM((_B_PER_W,), jnp.int32),
            pltpu.VMEM((_B_PER_W, E), jnp.float32),
            pltpu.SemaphoreType.DMA,
        ],
        compiler_params=pltpu.CompilerParams(use_tc_tiling_on_sc=False),
    )
    def gather_kernel(idx_hbm, table_hbm, out_hbm, idx_v, rows_v, sem):
        wid = lax.axis_index("s") * _NC + lax.axis_index("c")
        base = wid * _B_PER_W
        pltpu.sync_copy(idx_hbm.at[pl.ds(base, _B_PER_W)], idx_v)
        pltpu.async_copy(table_hbm.at[idx_v], rows_v, sem).wait()
        pltpu.sync_copy(rows_v, out_hbm.at[pl.ds(base, _B_PER_W)])

    return gather_kernel(idx, table)


def _proj_kernel(emb_ref, w_ref, b_ref, out_ref):
    acc = lax.dot_general(
        emb_ref[...],
        w_ref[...],
        dimension_numbers=(((1,), (1,)), ((), ())),
        preferred_element_type=jnp.float32,
    )
    out_ref[...] = acc + b_ref[...]


def _tc_project(emb, w, b2):
    return pl.pallas_call(
        _proj_kernel,
        grid=(GRID_V,),
        in_specs=[
            pl.BlockSpec((B, E), lambda j: (0, 0)),
            pl.BlockSpec((TILE_V, E), lambda j: (j, 0)),
            pl.BlockSpec((1, TILE_V), lambda j: (0, j)),
        ],
        out_specs=pl.BlockSpec((B, TILE_V), lambda j: (0, j)),
        out_shape=jax.ShapeDtypeStruct((B, V), jnp.float32),
    )(emb, w, b2)


def kernel(target, emb_table, W0, b0):
    idx = target.astype(jnp.int32)
    emb = jnp.take(emb_table, idx, axis=0)
    out = _tc_project(emb, W0, b0.reshape(1, V))
    return out[:, None, :]


# speedup vs baseline: 2.3626x; 1.1130x over previous
import jax
import jax.numpy as jnp
from jax import lax
from jax.experimental import pallas as pl
from jax.experimental.pallas import tpu as pltpu

V = 100000
E = 16
B = 1024

TILE_V = 2048
GRID_V = (V + TILE_V - 1) // TILE_V


def _gather_kernel(idx_ref, table_ref, emb_ref):
    def body(i, _):
        v = idx_ref[i]
        emb_ref[pl.ds(i, 1), :] = table_ref[pl.ds(v, 1), :]
        return 0

    lax.fori_loop(0, B, body, 0)


def _tc_gather(idx, table):
    return pl.pallas_call(
        _gather_kernel,
        in_specs=[
            pl.BlockSpec(memory_space=pltpu.SMEM),
            pl.BlockSpec((V, E), lambda: (0, 0)),
        ],
        out_specs=pl.BlockSpec((B, E), lambda: (0, 0)),
        out_shape=jax.ShapeDtypeStruct((B, E), jnp.float32),
    )(idx, table)


def _proj_kernel(emb_ref, w_ref, b_ref, out_ref):
    acc = lax.dot_general(
        emb_ref[...],
        w_ref[...],
        dimension_numbers=(((1,), (1,)), ((), ())),
        preferred_element_type=jnp.float32,
    )
    out_ref[...] = acc + b_ref[...][None, :]


def _tc_project(emb, w, b):
    return pl.pallas_call(
        _proj_kernel,
        grid=(GRID_V,),
        in_specs=[
            pl.BlockSpec((B, E), lambda j: (0, 0)),
            pl.BlockSpec((TILE_V, E), lambda j: (j, 0)),
            pl.BlockSpec((TILE_V,), lambda j: (j,)),
        ],
        out_specs=pl.BlockSpec((B, TILE_V), lambda j: (0, j)),
        out_shape=jax.ShapeDtypeStruct((B, V), jnp.float32),
    )(emb, w, b)


def kernel(target, emb_table, W0, b0):
    idx = target.astype(jnp.int32)
    emb = _tc_gather(idx, emb_table)
    out = _tc_project(emb, W0, b0)
    return out[:, None, :]
